# baseline (speedup 1.0000x reference)
import functools

import jax
import jax.numpy as jnp
from jax import lax
from jax.experimental import pallas as pl
from jax.experimental.pallas import tpu as pltpu
from jax.experimental.pallas import tpu_sc as plsc

N = 10000
D = 128
NPAD = 10240
DUMMY = N
NC = 2
NS = 16
NW = NC * NS
CHUNK = 64
NEG = -1e30
EPS = 1e-30


def _mm_att(xp, W, am):
    BLK = 256

    def body(x_ref, w_ref, am_ref, h_ref, a_ref):
        h = jnp.dot(x_ref[...], w_ref[...], preferred_element_type=jnp.float32)
        h_ref[...] = h
        a_ref[...] = jnp.dot(h, am_ref[...], preferred_element_type=jnp.float32)

    return pl.pallas_call(
        body,
        grid=(NPAD // BLK,),
        in_specs=[pl.BlockSpec((BLK, D), lambda i: (i, 0)),
                  pl.BlockSpec((D, D), lambda i: (0, 0)),
                  pl.BlockSpec((D, 128), lambda i: (0, 0))],
        out_specs=[pl.BlockSpec((BLK, D), lambda i: (i, 0)),
                   pl.BlockSpec((BLK, 128), lambda i: (i, 0))],
        out_shape=[jax.ShapeDtypeStruct((NPAD, D), jnp.float32),
                   jax.ShapeDtypeStruct((NPAD, 128), jnp.float32)],
    )(xp, W, am)


def _norm_mm_att(accp, denp, b, W, am):
    BLK = 256

    def body(acc_ref, den_ref, b_ref, w_ref, am_ref, h_ref, a_ref):
        den = jnp.sum(den_ref[...], axis=1)
        x = (acc_ref[0] + acc_ref[1]) / (den + EPS)[:, None] + b_ref[...]
        x = jnp.maximum(x, 0.0)
        h = jnp.dot(x, w_ref[...], preferred_element_type=jnp.float32)
        h_ref[...] = h
        a_ref[...] = jnp.dot(h, am_ref[...], preferred_element_type=jnp.float32)

    return pl.pallas_call(
        body,
        grid=(NPAD // BLK,),
        in_specs=[pl.BlockSpec((2, BLK, D), lambda i: (0, i, 0)),
                  pl.BlockSpec((BLK, NW), lambda i: (i, 0)),
                  pl.BlockSpec((1, D), lambda i: (0, 0)),
                  pl.BlockSpec((D, D), lambda i: (0, 0)),
                  pl.BlockSpec((D, 128), lambda i: (0, 0))],
        out_specs=[pl.BlockSpec((BLK, D), lambda i: (i, 0)),
                   pl.BlockSpec((BLK, 128), lambda i: (i, 0))],
        out_shape=[jax.ShapeDtypeStruct((NPAD, D), jnp.float32),
                   jax.ShapeDtypeStruct((NPAD, 128), jnp.float32)],
    )(accp, denp, b, W, am)


def _finalize(accp, denp, b):
    BLK = 400

    def body(acc_ref, den_ref, b_ref, o_ref):
        den = jnp.sum(den_ref[...], axis=1)
        o_ref[...] = (acc_ref[0] + acc_ref[1]) / (den + EPS)[:, None] + b_ref[...]

    return pl.pallas_call(
        body,
        grid=(N // BLK,),
        in_specs=[pl.BlockSpec((2, BLK, D), lambda i: (0, i, 0)),
                  pl.BlockSpec((BLK, NW), lambda i: (i, 0)),
                  pl.BlockSpec((1, D), lambda i: (0, 0))],
        out_specs=pl.BlockSpec((BLK, D), lambda i: (i, 0)),
        out_shape=jax.ShapeDtypeStruct((N, D), jnp.float32),
    )(accp, denp, b)


def _sc_edge(h, asrc, adst, mv, ei3d):
    nch = ei3d.shape[0]
    cpt = nch // NW
    rpt = NPAD // NS
    mesh = plsc.VectorSubcoreMesh(core_axis_name="c", subcore_axis_name="s")

    @functools.partial(
        pl.kernel,
        mesh=mesh,
        out_type=[jax.ShapeDtypeStruct((NC, NPAD, D), jnp.float32),
                  jax.ShapeDtypeStruct((NW, NPAD), jnp.float32)],
        scratch_types=[
            pltpu.VMEM((NPAD,), jnp.float32),
            pltpu.VMEM((NPAD,), jnp.float32),
            pltpu.VMEM((NPAD,), jnp.float32),
            pltpu.VMEM((2, 2, CHUNK), jnp.int32),
            pltpu.VMEM((2, CHUNK), jnp.int32),
            pltpu.VMEM((2, CHUNK), jnp.float32),
            pltpu.VMEM((16,), jnp.float32),
            pltpu.VMEM((2, CHUNK, D), jnp.float32),
            pltpu.VMEM_SHARED((NPAD, D), jnp.float32),
            pltpu.SemaphoreType.DMA,
            pltpu.SemaphoreType.DMA,
            pltpu.SemaphoreType.DMA,
            pltpu.SemaphoreType.DMA,
            pltpu.SemaphoreType.DMA,
            pltpu.SemaphoreType.DMA,
        ],
        compiler_params=pltpu.CompilerParams(needs_layout_passes=False),
    )
    def k(h_hbm, asrc_hbm, adst_hbm, m_hbm, ei_hbm,
          acc_hbm, den_hbm,
          asrc_t, adst_t, den_t, eb, sb, p_b, m_b, rows_b, acc_sh,
          gsem0, gsem1, isem0, isem1, ssem0, ssem1):
        c = lax.axis_index("c")
        s = lax.axis_index("s")
        w = s * NC + c
        gsem = (gsem0, gsem1)
        isem = (isem0, isem1)
        ssem = (ssem0, ssem1)

        pltpu.sync_copy(asrc_hbm, asrc_t)
        pltpu.sync_copy(adst_hbm, adst_t)
        pltpu.sync_copy(m_hbm, m_b)
        m_v = m_b[...]

        def scalar_stage(b):
            for g in range(CHUNK // 16):
                sl = pl.ds(g * 16, 16)
                s16 = eb[b, 0, sl]
                d16 = eb[b, 1, sl]
                z = plsc.load_gather(asrc_t, [s16]) + plsc.load_gather(adst_t, [d16])
                e = jnp.maximum(z, 0.2 * z)
                p = jnp.exp(e - m_v)
                p_b[b, sl] = p
                plsc.addupdate_scatter(den_t, [d16], p)

        def start_gather(b):
            pltpu.make_async_copy(h_hbm.at[eb.at[b, 0]], rows_b.at[b],
                                  gsem[b]).start()

        def wait_gather(b):
            pltpu.make_async_copy(h_hbm.at[eb.at[b, 0]], rows_b.at[b],
                                  gsem[b]).wait()

        def start_idx(b, i):
            pltpu.make_async_copy(ei_hbm.at[w * cpt + i], eb.at[b],
                                  isem[b]).start()

        def wait_idx(b, i):
            pltpu.make_async_copy(ei_hbm.at[w * cpt + i], eb.at[b],
                                  isem[b]).wait()

        def start_scatter(b):
            for g in range(CHUNK // 16):
                sl = pl.ds(g * 16, 16)
                sb[b, sl] = eb[b, 1, sl]
            pltpu.make_async_copy(rows_b.at[b], acc_sh.at[sb.at[b]],
                                  ssem[b]).start(add=True)

        def wait_scatter(b):
            pltpu.make_async_copy(rows_b.at[b], acc_sh.at[sb.at[b]],
                                  ssem[b]).wait()

        UNROLL = 8

        def scale(b):
            def srow(g, _):
                for u in range(UNROLL):
                    j = g * UNROLL + u
                    pj = plsc.load_gather(
                        p_b, [jnp.full((16,), b, jnp.int32),
                              jnp.full((16,), j, jnp.int32)])
                    for q in range(D // 16):
                        sl2 = pl.ds(q * 16, 16)
                        rows_b[b, j, sl2] = rows_b[b, j, sl2] * pj
                return 0
            lax.fori_loop(0, CHUNK // UNROLL, srow, 0)

        pltpu.sync_copy(ei_hbm.at[w * cpt], eb.at[0])
        start_gather(0)
        if cpt > 1:
            start_idx(1, 1)

        z16 = jnp.zeros((16,), jnp.float32)

        def zden(i, _):
            den_t[pl.ds(i * 16, 16)] = z16
            return 0
        lax.fori_loop(0, NPAD // 16, zden, 0)

        def zrow(i, _):
            for q in range(D // 16):
                rows_b[1, i, pl.ds(q * 16, 16)] = z16
            return 0
        lax.fori_loop(0, CHUNK, zrow, 0)

        for j in range(rpt // CHUNK):
            pltpu.make_async_copy(
                rows_b.at[1],
                acc_sh.at[pl.ds(s * rpt + j * CHUNK, CHUNK)], ssem0).start()
        scalar_stage(0)
        for j in range(rpt // CHUNK):
            pltpu.make_async_copy(
                rows_b.at[1],
                acc_sh.at[pl.ds(s * rpt + j * CHUNK, CHUNK)], ssem0).wait()
        plsc.subcore_barrier()

        def pair(i2, _):
            for b in range(2):
                i = i2 * 2 + b
                nb = 1 - b

                @pl.when((i + 1 < cpt) & (i > 0))
                def _():
                    wait_scatter(nb)

                @pl.when(i + 1 < cpt)
                def _():
                    wait_idx(nb, i + 1)
                    start_gather(nb)
                    scalar_stage(nb)
                wait_gather(b)
                scale(b)
                start_scatter(b)

                @pl.when(i + 2 < cpt)
                def _():
                    start_idx(b, i + 2)
            return 0
        lax.fori_loop(0, (cpt + 1) // 2, pair, 0)
        wait_scatter(0)
        wait_scatter(1)

        plsc.subcore_barrier()
        for j in range(rpt // CHUNK):
            sl = pl.ds(s * rpt + j * CHUNK, CHUNK)
            pltpu.make_async_copy(acc_sh.at[sl], acc_hbm.at[c].at[sl],
                                  gsem0).start()
        pltpu.sync_copy(den_t, den_hbm.at[w])
        for j in range(rpt // CHUNK):
            sl = pl.ds(s * rpt + j * CHUNK, CHUNK)
            pltpu.make_async_copy(acc_sh.at[sl], acc_hbm.at[c].at[sl],
                                  gsem0).wait()

    return k(h, asrc, adst, mv, ei3d)


def kernel(x, edge_index, W1, att_src1, att_dst1, b1, W2, att_src2, att_dst2, b2):
    n = x.shape[0]
    xp = jnp.zeros((NPAD, D), jnp.float32).at[:n].set(x)

    loop = jnp.arange(n, dtype=edge_index.dtype)
    src = jnp.concatenate([edge_index[0], loop])
    dst = jnp.concatenate([edge_index[1], loop])
    e_tot = src.shape[0]
    epad = -(-e_tot // (2 * NW * CHUNK)) * (2 * NW * CHUNK)
    src2d = jnp.full((epad,), DUMMY, jnp.int32).at[:e_tot].set(src).reshape(-1, 1, CHUNK)
    dst2d = jnp.full((epad,), DUMMY, jnp.int32).at[:e_tot].set(dst).reshape(-1, 1, CHUNK)
    ei3d = jnp.concatenate([src2d, dst2d], axis=1)

    am1 = jnp.zeros((D, 128), jnp.float32).at[:, 0].set(att_src1).at[:, 1].set(att_dst1)
    am2 = jnp.zeros((D, 128), jnp.float32).at[:, 0].set(att_src2).at[:, 1].set(att_dst2)

    h1, a1 = _mm_att(xp, W1, am1)
    asrc1 = a1[:, 0].at[n:].set(NEG)
    adst1 = a1[:, 1].at[n:].set(NEG)
    m1 = jnp.maximum(jnp.max(a1[:n, 0]) + jnp.max(a1[:n, 1]), 0.0)
    acc1, den1 = _sc_edge(h1, asrc1, adst1, jnp.full((16,), m1, jnp.float32),
                          ei3d)

    h2, a2 = _norm_mm_att(acc1, den1.T, b1.reshape(1, D), W2, am2)
    asrc2 = a2[:, 0].at[n:].set(NEG)
    adst2 = a2[:, 1].at[n:].set(NEG)
    m2 = jnp.maximum(jnp.max(a2[:n, 0]) + jnp.max(a2[:n, 1]), 0.0)
    acc2, den2 = _sc_edge(h2, asrc2, adst2, jnp.full((16,), m2, jnp.float32),
                          ei3d)

    return _finalize(acc2, den2.T, b2.reshape(1, D))


# speedup vs baseline: 1.0144x; 1.0144x over previous
import functools

import jax
import jax.numpy as jnp
from jax import lax
from jax.experimental import pallas as pl
from jax.experimental.pallas import tpu as pltpu
from jax.experimental.pallas import tpu_sc as plsc

N = 10000
D = 128
NPAD = 10240
DUMMY = N
NC = 2
NS = 16
NW = NC * NS
CHUNK = 64
NEG = -1e30
EPS = 1e-30


def _score_tail(a, i, BLK):
    rid = i * BLK + lax.broadcasted_iota(jnp.int32, (BLK, 2), 0)
    return jnp.where(rid < N, a, NEG)


def _mm_att(xp, W, am):
    BLK = 256

    def body(x_ref, w_ref, am_ref, h_ref, a_ref):
        h = jnp.dot(x_ref[...], w_ref[...], preferred_element_type=jnp.float32)
        h_ref[...] = h
        a = jnp.dot(h, am_ref[...], preferred_element_type=jnp.float32)
        a_ref[...] = _score_tail(a, pl.program_id(0), BLK)

    return pl.pallas_call(
        body,
        grid=(NPAD // BLK,),
        in_specs=[pl.BlockSpec((BLK, D), lambda i: (i, 0)),
                  pl.BlockSpec((D, D), lambda i: (0, 0)),
                  pl.BlockSpec((D, 2), lambda i: (0, 0))],
        out_specs=[pl.BlockSpec((BLK, D), lambda i: (i, 0)),
                   pl.BlockSpec((BLK, 2), lambda i: (i, 0))],
        out_shape=[jax.ShapeDtypeStruct((NPAD, D), jnp.float32),
                   jax.ShapeDtypeStruct((NPAD, 2), jnp.float32)],
    )(xp, W, am)


def _norm_mm_att(accp, denp, b, W, am):
    BLK = 256

    def body(acc_ref, den_ref, b_ref, w_ref, am_ref, h_ref, a_ref):
        den = jnp.sum(den_ref[...], axis=0)
        x = (acc_ref[0] + acc_ref[1]) / (den + EPS)[:, None] + b_ref[...]
        x = jnp.maximum(x, 0.0)
        h = jnp.dot(x, w_ref[...], preferred_element_type=jnp.float32)
        h_ref[...] = h
        a = jnp.dot(h, am_ref[...], preferred_element_type=jnp.float32)
        a_ref[...] = _score_tail(a, pl.program_id(0), BLK)

    return pl.pallas_call(
        body,
        grid=(NPAD // BLK,),
        in_specs=[pl.BlockSpec((2, BLK, D), lambda i: (0, i, 0)),
                  pl.BlockSpec((NW, BLK), lambda i: (0, i)),
                  pl.BlockSpec((1, D), lambda i: (0, 0)),
                  pl.BlockSpec((D, D), lambda i: (0, 0)),
                  pl.BlockSpec((D, 2), lambda i: (0, 0))],
        out_specs=[pl.BlockSpec((BLK, D), lambda i: (i, 0)),
                   pl.BlockSpec((BLK, 2), lambda i: (i, 0))],
        out_shape=[jax.ShapeDtypeStruct((NPAD, D), jnp.float32),
                   jax.ShapeDtypeStruct((NPAD, 2), jnp.float32)],
    )(accp, denp, b, W, am)


def _finalize(accp, denp, b):
    BLK = 400

    def body(acc_ref, den_ref, b_ref, o_ref):
        den = jnp.sum(den_ref[...], axis=1)
        o_ref[...] = (acc_ref[0] + acc_ref[1]) / (den + EPS)[:, None] + b_ref[...]

    return pl.pallas_call(
        body,
        grid=(N // BLK,),
        in_specs=[pl.BlockSpec((2, BLK, D), lambda i: (0, i, 0)),
                  pl.BlockSpec((BLK, NW), lambda i: (i, 0)),
                  pl.BlockSpec((1, D), lambda i: (0, 0))],
        out_specs=pl.BlockSpec((BLK, D), lambda i: (i, 0)),
        out_shape=jax.ShapeDtypeStruct((N, D), jnp.float32),
    )(accp, denp, b)


def _sc_edge(h, asrc, adst, mv, ei3d):
    nch = ei3d.shape[0]
    cpt = nch // NW
    rpt = NPAD // NS
    mesh = plsc.VectorSubcoreMesh(core_axis_name="c", subcore_axis_name="s")

    @functools.partial(
        pl.kernel,
        mesh=mesh,
        out_type=[jax.ShapeDtypeStruct((NC, NPAD, D), jnp.float32),
                  jax.ShapeDtypeStruct((NW, NPAD), jnp.float32)],
        scratch_types=[
            pltpu.VMEM((NPAD,), jnp.float32),
            pltpu.VMEM((NPAD,), jnp.float32),
            pltpu.VMEM((NPAD,), jnp.float32),
            pltpu.VMEM((2, 2, CHUNK), jnp.int32),
            pltpu.VMEM((2, CHUNK), jnp.int32),
            pltpu.VMEM((2, CHUNK), jnp.float32),
            pltpu.VMEM((16,), jnp.float32),
            pltpu.VMEM((2, CHUNK, D), jnp.float32),
            pltpu.VMEM_SHARED((NPAD, D), jnp.float32),
            pltpu.SemaphoreType.DMA,
            pltpu.SemaphoreType.DMA,
            pltpu.SemaphoreType.DMA,
            pltpu.SemaphoreType.DMA,
            pltpu.SemaphoreType.DMA,
            pltpu.SemaphoreType.DMA,
        ],
        compiler_params=pltpu.CompilerParams(needs_layout_passes=False),
    )
    def k(h_hbm, asrc_hbm, adst_hbm, m_hbm, ei_hbm,
          acc_hbm, den_hbm,
          asrc_t, adst_t, den_t, eb, sb, p_b, m_b, rows_b, acc_sh,
          gsem0, gsem1, isem0, isem1, ssem0, ssem1):
        c = lax.axis_index("c")
        s = lax.axis_index("s")
        w = s * NC + c
        gsem = (gsem0, gsem1)
        isem = (isem0, isem1)
        ssem = (ssem0, ssem1)

        pltpu.sync_copy(asrc_hbm, asrc_t)
        pltpu.sync_copy(adst_hbm, adst_t)
        pltpu.sync_copy(m_hbm, m_b)
        m_v = m_b[...]

        def scalar_stage(b):
            for g in range(CHUNK // 16):
                sl = pl.ds(g * 16, 16)
                s16 = eb[b, 0, sl]
                d16 = eb[b, 1, sl]
                z = plsc.load_gather(asrc_t, [s16]) + plsc.load_gather(adst_t, [d16])
                e = jnp.maximum(z, 0.2 * z)
                p = jnp.exp(e - m_v)
                p_b[b, sl] = p
                plsc.addupdate_scatter(den_t, [d16], p)

        def start_gather(b):
            pltpu.make_async_copy(h_hbm.at[eb.at[b, 0]], rows_b.at[b],
                                  gsem[b]).start()

        def wait_gather(b):
            pltpu.make_async_copy(h_hbm.at[eb.at[b, 0]], rows_b.at[b],
                                  gsem[b]).wait()

        def start_idx(b, i):
            pltpu.make_async_copy(ei_hbm.at[w * cpt + i], eb.at[b],
                                  isem[b]).start()

        def wait_idx(b, i):
            pltpu.make_async_copy(ei_hbm.at[w * cpt + i], eb.at[b],
                                  isem[b]).wait()

        def start_scatter(b):
            for g in range(CHUNK // 16):
                sl = pl.ds(g * 16, 16)
                sb[b, sl] = eb[b, 1, sl]
            pltpu.make_async_copy(rows_b.at[b], acc_sh.at[sb.at[b]],
                                  ssem[b]).start(add=True)

        def wait_scatter(b):
            pltpu.make_async_copy(rows_b.at[b], acc_sh.at[sb.at[b]],
                                  ssem[b]).wait()

        UNROLL = 8

        def scale(b):
            def srow(g, _):
                for u in range(UNROLL):
                    j = g * UNROLL + u
                    pj = plsc.load_gather(
                        p_b, [jnp.full((16,), b, jnp.int32),
                              jnp.full((16,), j, jnp.int32)])
                    for q in range(D // 16):
                        sl2 = pl.ds(q * 16, 16)
                        rows_b[b, j, sl2] = rows_b[b, j, sl2] * pj
                return 0
            lax.fori_loop(0, CHUNK // UNROLL, srow, 0)

        pltpu.sync_copy(ei_hbm.at[w * cpt], eb.at[0])
        start_gather(0)
        if cpt > 1:
            start_idx(1, 1)

        z16 = jnp.zeros((16,), jnp.float32)

        def zden(i, _):
            den_t[pl.ds(i * 16, 16)] = z16
            return 0
        lax.fori_loop(0, NPAD // 16, zden, 0)

        def zrow(i, _):
            for q in range(D // 16):
                rows_b[1, i, pl.ds(q * 16, 16)] = z16
            return 0
        lax.fori_loop(0, CHUNK, zrow, 0)

        for j in range(rpt // CHUNK):
            pltpu.make_async_copy(
                rows_b.at[1],
                acc_sh.at[pl.ds(s * rpt + j * CHUNK, CHUNK)], ssem0).start()
        scalar_stage(0)
        for j in range(rpt // CHUNK):
            pltpu.make_async_copy(
                rows_b.at[1],
                acc_sh.at[pl.ds(s * rpt + j * CHUNK, CHUNK)], ssem0).wait()
        plsc.subcore_barrier()

        def pair(i2, _):
            for b in range(2):
                i = i2 * 2 + b
                nb = 1 - b

                @pl.when((i + 1 < cpt) & (i > 0))
                def _():
                    wait_scatter(nb)

                @pl.when(i + 1 < cpt)
                def _():
                    wait_idx(nb, i + 1)
                    start_gather(nb)
                    scalar_stage(nb)
                wait_gather(b)
                scale(b)
                start_scatter(b)

                @pl.when(i + 2 < cpt)
                def _():
                    start_idx(b, i + 2)
            return 0
        lax.fori_loop(0, (cpt + 1) // 2, pair, 0)
        wait_scatter(0)
        wait_scatter(1)

        plsc.subcore_barrier()
        for j in range(rpt // CHUNK):
            sl = pl.ds(s * rpt + j * CHUNK, CHUNK)
            pltpu.make_async_copy(acc_sh.at[sl], acc_hbm.at[c].at[sl],
                                  gsem0).start()
        pltpu.sync_copy(den_t, den_hbm.at[w])
        for j in range(rpt // CHUNK):
            sl = pl.ds(s * rpt + j * CHUNK, CHUNK)
            pltpu.make_async_copy(acc_sh.at[sl], acc_hbm.at[c].at[sl],
                                  gsem0).wait()

    return k(h, asrc, adst, mv, ei3d)


def kernel(x, edge_index, W1, att_src1, att_dst1, b1, W2, att_src2, att_dst2, b2):
    n = x.shape[0]
    xp = jnp.zeros((NPAD, D), jnp.float32).at[:n].set(x)

    loop = jnp.arange(n, dtype=edge_index.dtype)
    src = jnp.concatenate([edge_index[0], loop])
    dst = jnp.concatenate([edge_index[1], loop])
    e_tot = src.shape[0]
    epad = -(-e_tot // (2 * NW * CHUNK)) * (2 * NW * CHUNK)
    src2d = jnp.full((epad,), DUMMY, jnp.int32).at[:e_tot].set(src).reshape(-1, 1, CHUNK)
    dst2d = jnp.full((epad,), DUMMY, jnp.int32).at[:e_tot].set(dst).reshape(-1, 1, CHUNK)
    ei3d = jnp.concatenate([src2d, dst2d], axis=1)

    am1 = jnp.stack([att_src1, att_dst1], axis=1)
    am2 = jnp.stack([att_src2, att_dst2], axis=1)

    h1, a1 = _mm_att(xp, W1, am1)
    m1 = jnp.maximum(jnp.max(a1[:, 0]) + jnp.max(a1[:, 1]), 0.0)
    acc1, den1 = _sc_edge(h1, a1[:, 0], a1[:, 1],
                          jnp.full((16,), m1, jnp.float32), ei3d)

    h2, a2 = _norm_mm_att(acc1, den1, b1.reshape(1, D), W2, am2)
    m2 = jnp.maximum(jnp.max(a2[:, 0]) + jnp.max(a2[:, 1]), 0.0)
    acc2, den2 = _sc_edge(h2, a2[:, 0], a2[:, 1],
                          jnp.full((16,), m2, jnp.float32), ei3d)

    return _finalize(acc2, den2.T, b2.reshape(1, D))
